# baseline (speedup 1.0000x reference)
import functools

import jax
import jax.numpy as jnp
from jax import lax
from jax.experimental import pallas as pl
from jax.experimental.pallas import tpu as pltpu
from jax.experimental.pallas import tpu_sc as plsc

_L = 16
_CHUNK = 128


def _softplus(x):
    y = jnp.exp(-jnp.abs(x))
    s = y / (2.0 + y)
    s2 = s * s
    p = 2.0 * s * (1.0 + s2 * (1.0 / 3.0 + s2 * (0.2 + s2 * (1.0 / 7.0))))
    return jnp.maximum(x, 0.0) + p


@functools.lru_cache(maxsize=None)
def _build(B, V, D, NEG_K):
    info = plsc.get_sparse_core_info()
    NC, NS = info.num_cores, info.num_subcores
    NW = NC * NS
    assert B % (NW * _CHUNK) == 0 and NEG_K == 2 and D == 64 and V % 2 == 0
    b_per_w = B // NW
    n_sub = b_per_w // _CHUNK
    W = 2 * D

    mesh = plsc.VectorSubcoreMesh(core_axis_name="c", subcore_axis_name="s")

    @functools.partial(
        pl.kernel,
        mesh=mesh,
        compiler_params=pltpu.CompilerParams(
            needs_layout_passes=False, use_tc_tiling_on_sc=True),
        out_type=jax.ShapeDtypeStruct((B,), jnp.float32),
        scratch_types=[
            pltpu.VMEM((_CHUNK,), jnp.int32),
            pltpu.VMEM((_CHUNK,), jnp.int32),
            pltpu.VMEM((_CHUNK,), jnp.int32),
            pltpu.VMEM((_CHUNK,), jnp.int32),
            pltpu.VMEM((_CHUNK,), jnp.int32),
            pltpu.VMEM((_CHUNK,), jnp.int32),
            pltpu.VMEM((_CHUNK,), jnp.int32),
            pltpu.VMEM((_CHUNK,), jnp.int32),
            pltpu.VMEM((_CHUNK, 2 * 64), jnp.float32),
            pltpu.VMEM((_CHUNK, 2 * 64), jnp.float32),
            pltpu.VMEM((_CHUNK, 2 * 64), jnp.float32),
            pltpu.VMEM((_CHUNK, 2 * 64), jnp.float32),
            pltpu.VMEM((_CHUNK,), jnp.float32),
            pltpu.SemaphoreType.DMA,
        ],
    )
    def line_sc(i_hbm, j_hbm, neg_hbm, emb2_hbm, out_hbm,
                idx_i, idx_j, idx_n0, idx_n1,
                idh_i, idh_j, idh_n0, idh_n1,
                ri, rj, rn0, rn1, out_v, sem):
        wid = lax.axis_index("s") * NC + lax.axis_index("c")
        lanes = lax.iota(jnp.int32, 16)
        zero = jnp.zeros((_L,), jnp.float32)
        for sub in range(n_sub):
            base = wid * b_per_w + sub * _CHUNK
            pltpu.sync_copy(i_hbm.at[pl.ds(base, _CHUNK)], idx_i)
            pltpu.sync_copy(j_hbm.at[pl.ds(base, _CHUNK)], idx_j)
            pltpu.sync_copy(neg_hbm.at[0, pl.ds(base, _CHUNK)], idx_n0)
            pltpu.sync_copy(neg_hbm.at[1, pl.ds(base, _CHUNK)], idx_n1)

            def half_body(k, _):
                sl = pl.ds(k * _L, _L)
                idh_i[sl] = lax.shift_right_logical(idx_i[sl], 1)
                idh_j[sl] = lax.shift_right_logical(idx_j[sl], 1)
                idh_n0[sl] = lax.shift_right_logical(idx_n0[sl], 1)
                idh_n1[sl] = lax.shift_right_logical(idx_n1[sl], 1)
                return 0

            lax.fori_loop(0, _CHUNK // _L, half_body, 0)

            copies = [
                pltpu.make_async_copy(emb2_hbm.at[idh_i], ri, sem),
                pltpu.make_async_copy(emb2_hbm.at[idh_j], rj, sem),
                pltpu.make_async_copy(emb2_hbm.at[idh_n0], rn0, sem),
                pltpu.make_async_copy(emb2_hbm.at[idh_n1], rn1, sem),
            ]
            for cp in copies:
                cp.start()
            for cp in copies:
                cp.wait()

            def group_body(g, _):
                sl = pl.ds(g * _L, _L)
                rows = g * _L + lanes
                off_i = lax.shift_left(lax.bitwise_and(idx_i[sl], 1), 6)
                off_j = lax.shift_left(lax.bitwise_and(idx_j[sl], 1), 6)
                off_0 = lax.shift_left(lax.bitwise_and(idx_n0[sl], 1), 6)
                off_1 = lax.shift_left(lax.bitwise_and(idx_n1[sl], 1), 6)

                def d_body(d, accs):
                    ap, a0, a1 = accs
                    vi = plsc.load_gather(ri, [rows, off_i + d])
                    vj = plsc.load_gather(rj, [rows, off_j + d])
                    v0 = plsc.load_gather(rn0, [rows, off_0 + d])
                    v1 = plsc.load_gather(rn1, [rows, off_1 + d])
                    return (ap + vi * vj, a0 + vi * v0, a1 + vi * v1)

                ap, a0, a1 = lax.fori_loop(0, D, d_body, (zero, zero, zero))
                res = _softplus(-ap) + _softplus(a0) + _softplus(a1)
                out_v[sl] = res
                return 0

            lax.fori_loop(0, _CHUNK // _L, group_body, 0)
            pltpu.sync_copy(out_v, out_hbm.at[pl.ds(base, _CHUNK)])

    return line_sc


def kernel(i, j, neg_set, emb):
    B = i.shape[0]
    V, D = emb.shape
    fn = _build(B, V, D, neg_set.shape[0])
    emb2 = jnp.reshape(emb, (V // 2, 2 * D))
    return fn(i.astype(jnp.int32), j.astype(jnp.int32),
              neg_set.astype(jnp.int32), emb2)


# speedup vs baseline: 1.6769x; 1.6769x over previous
import functools

import jax
import jax.numpy as jnp
from jax import lax
from jax.experimental import pallas as pl
from jax.experimental.pallas import tpu as pltpu
from jax.experimental.pallas import tpu_sc as plsc

_L = 16
_CHUNK = 128


def _softplus(x):
    y = jnp.exp(-jnp.abs(x))
    s = y / (2.0 + y)
    s2 = s * s
    p = 2.0 * s * (1.0 + s2 * (1.0 / 3.0 + s2 * (0.2 + s2 * (1.0 / 7.0))))
    return jnp.maximum(x, 0.0) + p


@functools.lru_cache(maxsize=None)
def _build(B, V, D, NEG_K):
    info = plsc.get_sparse_core_info()
    NC, NS = info.num_cores, info.num_subcores
    NW = NC * NS
    assert B % (NW * _CHUNK) == 0 and NEG_K == 2
    b_per_w = B // NW
    n_sub = b_per_w // _CHUNK

    mesh = plsc.VectorSubcoreMesh(core_axis_name="c", subcore_axis_name="s")

    @functools.partial(
        pl.kernel,
        mesh=mesh,
        compiler_params=pltpu.CompilerParams(
            needs_layout_passes=False, use_tc_tiling_on_sc=False),
        out_type=jax.ShapeDtypeStruct((B,), jnp.float32),
        scratch_types=[
            pltpu.VMEM((2, _CHUNK), jnp.int32),
            pltpu.VMEM((2, _CHUNK), jnp.int32),
            pltpu.VMEM((2, _CHUNK), jnp.int32),
            pltpu.VMEM((2, _CHUNK), jnp.int32),
            pltpu.VMEM((2, _CHUNK, D), jnp.float32),
            pltpu.VMEM((2, _CHUNK, D), jnp.float32),
            pltpu.VMEM((2, _CHUNK, D), jnp.float32),
            pltpu.VMEM((2, _CHUNK, D), jnp.float32),
            pltpu.VMEM((_CHUNK * _L,), jnp.float32),
            pltpu.VMEM((_CHUNK * _L,), jnp.float32),
            pltpu.VMEM((_CHUNK * _L,), jnp.float32),
            pltpu.VMEM((_CHUNK,), jnp.float32),
            pltpu.SemaphoreType.DMA,
        ],
    )
    def line_sc(i_hbm, j_hbm, neg_hbm, emb_hbm, out_hbm,
                idx_i, idx_j, idx_n0, idx_n1, ri, rj, rn0, rn1,
                part_p, part_0, part_1, out_v, sem):
        wid = lax.axis_index("s") * NC + lax.axis_index("c")
        lanes = lax.iota(jnp.int32, 16)
        zero = jnp.zeros((_L,), jnp.float32)

        def stage(sub, buf):
            base = wid * b_per_w + sub * _CHUNK
            pltpu.sync_copy(i_hbm.at[pl.ds(base, _CHUNK)], idx_i.at[buf])
            pltpu.sync_copy(j_hbm.at[pl.ds(base, _CHUNK)], idx_j.at[buf])
            pltpu.sync_copy(neg_hbm.at[0, pl.ds(base, _CHUNK)],
                            idx_n0.at[buf])
            pltpu.sync_copy(neg_hbm.at[1, pl.ds(base, _CHUNK)],
                            idx_n1.at[buf])
            copies = [
                pltpu.make_async_copy(
                    emb_hbm.at[idx_i.at[buf]], ri.at[buf], sem),
                pltpu.make_async_copy(
                    emb_hbm.at[idx_j.at[buf]], rj.at[buf], sem),
                pltpu.make_async_copy(
                    emb_hbm.at[idx_n0.at[buf]], rn0.at[buf], sem),
                pltpu.make_async_copy(
                    emb_hbm.at[idx_n1.at[buf]], rn1.at[buf], sem),
            ]
            for cp in copies:
                cp.start()
            return copies

        def compute(sub, buf):
            rib, rjb, r0b, r1b = ri.at[buf], rj.at[buf], rn0.at[buf], rn1.at[buf]
            base = wid * b_per_w + sub * _CHUNK

            def row_body(r, _):
                vi = [rib[r, pl.ds(c * _L, _L)] for c in range(D // _L)]
                vj = [rjb[r, pl.ds(c * _L, _L)] for c in range(D // _L)]
                v0 = [r0b[r, pl.ds(c * _L, _L)] for c in range(D // _L)]
                v1 = [r1b[r, pl.ds(c * _L, _L)] for c in range(D // _L)]
                pp = zero
                p0 = zero
                p1 = zero
                for c in range(D // _L):
                    pp = pp + vi[c] * vj[c]
                    p0 = p0 + vi[c] * v0[c]
                    p1 = p1 + vi[c] * v1[c]
                part_p[pl.ds(r * _L, _L)] = pp
                part_0[pl.ds(r * _L, _L)] = p0
                part_1[pl.ds(r * _L, _L)] = p1
                return 0

            lax.fori_loop(0, _CHUNK, row_body, 0)

            def group_body(g, _):
                base_idx = (g * _L + lanes) * _L
                ap, a0, a1 = zero, zero, zero
                for k in range(_L):
                    ap = ap + plsc.load_gather(part_p, [base_idx + k])
                    a0 = a0 + plsc.load_gather(part_0, [base_idx + k])
                    a1 = a1 + plsc.load_gather(part_1, [base_idx + k])
                res = _softplus(-ap) + _softplus(a0) + _softplus(a1)
                out_v[pl.ds(g * _L, _L)] = res
                return 0

            lax.fori_loop(0, _CHUNK // _L, group_body, 0)
            pltpu.sync_copy(out_v, out_hbm.at[pl.ds(base, _CHUNK)])

        inflight = stage(0, 0)
        for sub in range(n_sub):
            for cp in inflight:
                cp.wait()
            if sub + 1 < n_sub:
                nxt = stage(sub + 1, (sub + 1) % 2)
            else:
                nxt = []
            compute(sub, sub % 2)
            inflight = nxt

    return line_sc


def kernel(i, j, neg_set, emb):
    B = i.shape[0]
    V, D = emb.shape
    fn = _build(B, V, D, neg_set.shape[0])
    return fn(i.astype(jnp.int32), j.astype(jnp.int32),
              neg_set.astype(jnp.int32), emb)


# speedup vs baseline: 1.7725x; 1.0570x over previous
import functools

import jax
import jax.numpy as jnp
from jax import lax
from jax.experimental import pallas as pl
from jax.experimental.pallas import tpu as pltpu
from jax.experimental.pallas import tpu_sc as plsc

_L = 16
_CHUNK = 128


def _softplus(x):
    y = jnp.exp(-jnp.abs(x))
    s = y / (2.0 + y)
    s2 = s * s
    p = 2.0 * s * (1.0 + s2 * (1.0 / 3.0 + s2 * (0.2 + s2 * (1.0 / 7.0))))
    return jnp.maximum(x, 0.0) + p


@functools.lru_cache(maxsize=None)
def _build(B, V, D, NEG_K):
    info = plsc.get_sparse_core_info()
    NC, NS = info.num_cores, info.num_subcores
    NW = NC * NS
    assert B % (NW * _CHUNK) == 0 and NEG_K == 2
    b_per_w = B // NW
    n_sub = b_per_w // _CHUNK

    mesh = plsc.VectorSubcoreMesh(core_axis_name="c", subcore_axis_name="s")

    @functools.partial(
        pl.kernel,
        mesh=mesh,
        compiler_params=pltpu.CompilerParams(
            needs_layout_passes=False, use_tc_tiling_on_sc=False),
        out_type=jax.ShapeDtypeStruct((B,), jnp.float32),
        scratch_types=[
            pltpu.VMEM((b_per_w,), jnp.int32),
            pltpu.VMEM((b_per_w,), jnp.int32),
            pltpu.VMEM((b_per_w,), jnp.int32),
            pltpu.VMEM((b_per_w,), jnp.int32),
            pltpu.VMEM((2, _CHUNK, D), jnp.float32),
            pltpu.VMEM((2, _CHUNK, D), jnp.float32),
            pltpu.VMEM((2, _CHUNK, D), jnp.float32),
            pltpu.VMEM((2, _CHUNK, D), jnp.float32),
            pltpu.VMEM((_CHUNK * _L,), jnp.float32),
            pltpu.VMEM((_CHUNK * _L,), jnp.float32),
            pltpu.VMEM((_CHUNK * _L,), jnp.float32),
            pltpu.VMEM((_CHUNK,), jnp.float32),
            pltpu.SemaphoreType.DMA,
        ],
    )
    def line_sc(i_hbm, j_hbm, neg_hbm, emb_hbm, out_hbm,
                idx_i, idx_j, idx_n0, idx_n1, ri, rj, rn0, rn1,
                part_p, part_0, part_1, out_v, sem):
        wid = lax.axis_index("s") * NC + lax.axis_index("c")
        wbase = wid * b_per_w
        lanes = lax.iota(jnp.int32, 16)
        zero = jnp.zeros((_L,), jnp.float32)

        pltpu.sync_copy(i_hbm.at[pl.ds(wbase, b_per_w)], idx_i)
        pltpu.sync_copy(j_hbm.at[pl.ds(wbase, b_per_w)], idx_j)
        pltpu.sync_copy(neg_hbm.at[0, pl.ds(wbase, b_per_w)], idx_n0)
        pltpu.sync_copy(neg_hbm.at[1, pl.ds(wbase, b_per_w)], idx_n1)

        def stage(sub, buf):
            sl = pl.ds(sub * _CHUNK, _CHUNK)
            copies = [
                pltpu.make_async_copy(
                    emb_hbm.at[idx_i.at[sl]], ri.at[buf], sem),
                pltpu.make_async_copy(
                    emb_hbm.at[idx_j.at[sl]], rj.at[buf], sem),
                pltpu.make_async_copy(
                    emb_hbm.at[idx_n0.at[sl]], rn0.at[buf], sem),
                pltpu.make_async_copy(
                    emb_hbm.at[idx_n1.at[sl]], rn1.at[buf], sem),
            ]
            for cp in copies:
                cp.start()
            return copies

        def compute(sub, buf):
            rib, rjb, r0b, r1b = ri.at[buf], rj.at[buf], rn0.at[buf], rn1.at[buf]
            base = wbase + sub * _CHUNK

            def row_body(r, _):
                vi = [rib[r, pl.ds(c * _L, _L)] for c in range(D // _L)]
                vj = [rjb[r, pl.ds(c * _L, _L)] for c in range(D // _L)]
                v0 = [r0b[r, pl.ds(c * _L, _L)] for c in range(D // _L)]
                v1 = [r1b[r, pl.ds(c * _L, _L)] for c in range(D // _L)]
                pp = zero
                p0 = zero
                p1 = zero
                for c in range(D // _L):
                    pp = pp + vi[c] * vj[c]
                    p0 = p0 + vi[c] * v0[c]
                    p1 = p1 + vi[c] * v1[c]
                part_p[pl.ds(r * _L, _L)] = pp
                part_0[pl.ds(r * _L, _L)] = p0
                part_1[pl.ds(r * _L, _L)] = p1
                return 0

            lax.fori_loop(0, _CHUNK, row_body, 0)

            def group_body(g, _):
                base_idx = (g * _L + lanes) * _L
                ap, a0, a1 = zero, zero, zero
                for k in range(_L):
                    ap = ap + plsc.load_gather(part_p, [base_idx + k])
                    a0 = a0 + plsc.load_gather(part_0, [base_idx + k])
                    a1 = a1 + plsc.load_gather(part_1, [base_idx + k])
                res = _softplus(-ap) + _softplus(a0) + _softplus(a1)
                out_v[pl.ds(g * _L, _L)] = res
                return 0

            lax.fori_loop(0, _CHUNK // _L, group_body, 0)
            pltpu.sync_copy(out_v, out_hbm.at[pl.ds(base, _CHUNK)])

        inflight = stage(0, 0)
        for sub in range(n_sub):
            for cp in inflight:
                cp.wait()
            if sub + 1 < n_sub:
                nxt = stage(sub + 1, (sub + 1) % 2)
            else:
                nxt = []
            compute(sub, sub % 2)
            inflight = nxt

    return line_sc


def kernel(i, j, neg_set, emb):
    B = i.shape[0]
    V, D = emb.shape
    fn = _build(B, V, D, neg_set.shape[0])
    return fn(i.astype(jnp.int32), j.astype(jnp.int32),
              neg_set.astype(jnp.int32), emb)
